# Initial kernel scaffold; baseline (speedup 1.0000x reference)
import jax
import jax.numpy as jnp
from jax.experimental import pallas as pl


def kernel(x, edge_index, Win, b_in, Wq, bq, Wk, bk, Wv, bv, Wo, bo, g1, b1, g2, b2, Wf1, bf1, Wf2, bf2):
    raise NotImplementedError("write your pallas kernel here")


# speedup vs baseline: 12.1050x; 12.1050x over previous
import functools

import jax
import jax.numpy as jnp
import numpy as np
from jax import lax
from jax.experimental import pallas as pl
from jax.experimental.pallas import tpu as pltpu
from jax.experimental.pallas import tpu_sc as plsc

N = 10000
E = 320000
HID = 128
H = 8
DH = 16
FFN = 256
L = 2

NW = 32
CH = 128
E2 = E + N
PER_W = 10368
E2P = PER_W * NW
NCHUNK = PER_W // CH
NP = 10240
RPT = NP // 16

R = 1000
GRID = N // R
RE = 1024
GRIDE = E2P // RE

NF = 9
NT = 3
NP2 = 10368
NTH = NP2 // NT
TRASH = NTH
ACCR = NTH + 8
NCH2 = E2P // CH


def _dot(a, b):
    return lax.dot_general(a, b, (((1,), (0,)), ((), ())),
                           precision=lax.Precision.HIGHEST,
                           preferred_element_type=jnp.float32)


def _ln(h, g, b):
    mu = jnp.mean(h, axis=1, keepdims=True)
    d = h - mu
    var = jnp.mean(d * d, axis=1, keepdims=True)
    return d * lax.rsqrt(var + 1e-5) * g + b


def _in_body(x_ref, w_ref, b_ref, o_ref):
    o_ref[...] = _dot(x_ref[...], w_ref[...]) + b_ref[...]


def _qkv_body(h_ref, g_ref, b_ref, w_ref, bias_ref, q_ref, kv_ref):
    hn = _ln(h_ref[...], g_ref[...], b_ref[...])
    qkv = _dot(hn, w_ref[...]) + bias_ref[...]
    q_ref[...] = qkv[:, :HID]
    kv_ref[...] = qkv[:, HID:]


def _edge_body(dst_ref, qd_ref, kvg_ref, b1_ref, b2_ref, out_ref,
               wvs_ref, acc_ref):
    j = pl.program_id(0)

    @pl.when(j == 0)
    def _():
        acc_ref[...] = jnp.zeros((NP2, NF * 16), jnp.float32)

    rid = j * RE + lax.broadcasted_iota(jnp.int32, (RE, 1), 0)
    qd = qd_ref[...]
    kvg = kvg_ref[...]
    scores = _dot(qd * kvg[:, :HID], b1_ref[...])
    e = jnp.exp(scores) * jnp.where(rid < E2, 1.0, 0.0)
    erep = _dot(e, b2_ref[...])
    realf = jnp.where(rid < E, 1.0, 0.0)
    wvs_ref[...] = jnp.concatenate(
        [kvg[:, HID:] * erep, e, realf,
         jnp.zeros((RE, 7), jnp.float32)], axis=1)

    def body(k, carry):
        d = dst_ref[0, 0, k]
        acc_ref[pl.ds(d, 1), :] = (acc_ref[pl.ds(d, 1), :]
                                   + wvs_ref[pl.ds(k, 1), :])
        return carry

    lax.fori_loop(0, RE, body, 0)

    @pl.when(j == GRIDE - 1)
    def _():
        out_ref[...] = acc_ref[...]


def _post_body(h_ref, n_ref, s_ref, wo_ref, bo_ref,
               g2_ref, b2_ref, wf1_ref, bf1_ref, wf2_ref, bf2_ref, o_ref):
    num = n_ref[...]
    s = s_ref[...]
    rec = jnp.concatenate(
        [jnp.broadcast_to(s[:, h:h + 1], (R, DH)) for h in range(H)], axis=1)
    deg = s[:, 8:9]
    agg = jnp.where(deg > 0.0, num / rec, 0.0)
    h2 = h_ref[...] + _dot(agg, wo_ref[...]) + bo_ref[...]
    hn = _ln(h2, g2_ref[...], b2_ref[...])
    f = _dot(hn, wf1_ref[...]) + bf1_ref[...]
    f = 0.5 * f * (1.0 + lax.erf(f * 0.7071067811865476))
    o_ref[...] = h2 + _dot(f, wf2_ref[...]) + bf2_ref[...]


def _row_spec(rows, cols):
    return pl.BlockSpec((rows, cols), lambda i: (i, 0))


def _full_spec(shape):
    nd = len(shape)
    return pl.BlockSpec(shape, lambda i: (0,) * nd)


def _call_in(x, wT, b2):
    return pl.pallas_call(
        _in_body,
        grid=(GRID,),
        in_specs=[_row_spec(R, HID), _full_spec((HID, HID)),
                  _full_spec((1, HID))],
        out_specs=_row_spec(R, HID),
        out_shape=jax.ShapeDtypeStruct((N, HID), jnp.float32),
    )(x, wT, b2)


def _call_qkv(h, g, b, wT, bias):
    return pl.pallas_call(
        _qkv_body,
        grid=(GRID,),
        in_specs=[_row_spec(R, HID), _full_spec((1, HID)), _full_spec((1, HID)),
                  _full_spec((HID, 3 * HID)), _full_spec((1, 3 * HID))],
        out_specs=[_row_spec(R, HID), _row_spec(R, 2 * HID)],
        out_shape=[jax.ShapeDtypeStruct((N, HID), jnp.float32),
                   jax.ShapeDtypeStruct((N, 2 * HID), jnp.float32)],
    )(h, g, b, wT, bias)


def _call_edge_compute(dst3, qd, kvg, b1m, b2m):
    return pl.pallas_call(
        _edge_body,
        grid=(GRIDE,),
        in_specs=[pl.BlockSpec((1, 1, RE), lambda j: (j, 0, 0),
                               memory_space=pltpu.SMEM),
                  _row_spec(RE, HID), _row_spec(RE, 2 * HID),
                  _full_spec((HID, H)), _full_spec((H, HID))],
        out_specs=_full_spec((NP2, NF * 16)),
        out_shape=jax.ShapeDtypeStruct((NP2, NF * 16), jnp.float32),
        scratch_shapes=[pltpu.VMEM((RE, NF * 16), jnp.float32),
                        pltpu.VMEM((NP2, NF * 16), jnp.float32)],
    )(dst3, qd, kvg, b1m, b2m)


def _call_post(h, n, s, woT, bo, g2, b2, wf1T, bf1, wf2T, bf2):
    return pl.pallas_call(
        _post_body,
        grid=(GRID,),
        in_specs=[_row_spec(R, HID), _row_spec(R, HID),
                  _row_spec(R, 16),
                  _full_spec((HID, HID)), _full_spec((1, HID)),
                  _full_spec((1, HID)), _full_spec((1, HID)),
                  _full_spec((HID, FFN)), _full_spec((1, FFN)),
                  _full_spec((FFN, HID)), _full_spec((1, HID))],
        out_specs=_row_spec(R, HID),
        out_shape=jax.ShapeDtypeStruct((N, HID), jnp.float32),
    )(h, n, s, woT, bo, g2, b2, wf1T, bf1, wf2T, bf2)


def _make_gather_kernel():
    mesh = plsc.VectorSubcoreMesh(core_axis_name="c", subcore_axis_name="s")

    @functools.partial(
        pl.kernel, mesh=mesh,
        out_type=[jax.ShapeDtypeStruct((E2P, HID), jnp.float32),
                  jax.ShapeDtypeStruct((E2P, 2 * HID), jnp.float32)],
        scratch_types=[
            pltpu.VMEM((CH,), jnp.int32),
            pltpu.VMEM((CH,), jnp.int32),
            pltpu.VMEM((CH, HID), jnp.float32),
            pltpu.VMEM((CH, 2 * HID), jnp.float32),
            pltpu.SemaphoreType.DMA,
            pltpu.SemaphoreType.DMA,
        ])
    def gather_kernel(q_hbm, kv_hbm, src_hbm, dst_hbm, qd_out, kvg_out,
                      srcb, dstb, qrows, kvrows, sem1, sem2):
        cid = lax.axis_index("c")
        sid = lax.axis_index("s")
        wid = sid * 2 + cid
        ebase = wid * PER_W

        def chunk(t, carry):
            base = ebase + t * CH
            pltpu.sync_copy(src_hbm.at[pl.ds(base, CH)], srcb)
            pltpu.sync_copy(dst_hbm.at[pl.ds(base, CH)], dstb)
            cq = pltpu.async_copy(q_hbm.at[dstb], qrows, sem1)
            ck = pltpu.async_copy(kv_hbm.at[srcb], kvrows, sem2)
            cq.wait()
            ck.wait()
            pltpu.sync_copy(qrows, qd_out.at[pl.ds(base, CH)])
            pltpu.sync_copy(kvrows, kvg_out.at[pl.ds(base, CH)])
            return carry

        lax.fori_loop(0, NCHUNK, chunk, 0)

    return gather_kernel


_gather_kernel = _make_gather_kernel()


def kernel(x, edge_index, Win, b_in, Wq, bq, Wk, bk, Wv, bv, Wo, bo,
           g1, b1, g2, b2, Wf1, bf1, Wf2, bf2):
    src = edge_index[0]
    dst = edge_index[1]
    idxN = jnp.arange(N, dtype=jnp.int32)
    pad = jnp.zeros((E2P - E2,), jnp.int32)
    src2 = jnp.concatenate([src, idxN, pad])
    dst2 = jnp.concatenate([dst, idxN, pad])
    blk = (np.arange(HID)[:, None] // DH) == np.arange(H)[None, :]
    b1m = jnp.asarray(blk, jnp.float32)
    b2m = jnp.asarray(blk.T, jnp.float32)

    h = _call_in(x, Win.T, b_in[None, :])
    inv = 1.0 / jnp.sqrt(float(DH))
    dst3 = dst2.reshape(GRIDE, 1, RE)
    for l in range(L):
        wqkvT = jnp.concatenate(
            [Wq[l].T * inv, Wk[l].T, Wv[l].T], axis=1)
        bqkv = jnp.concatenate([bq[l] * inv, bk[l], bv[l]])[None, :]
        q, kv = _call_qkv(h, g1[l][None, :], b1[l][None, :], wqkvT, bqkv)
        qd, kvg = _gather_kernel(q, kv, src2, dst2)
        acc2 = _call_edge_compute(dst3, qd, kvg, b1m, b2m)
        num = acc2[:N, :HID]
        ss = acc2[:N, HID:HID + 16]
        h = _call_post(h, num, ss,
                       Wo[l].T, bo[l][None, :], g2[l][None, :], b2[l][None, :],
                       Wf1[l].T, bf1[l][None, :], Wf2[l].T, bf2[l][None, :])
    return h


# speedup vs baseline: 15.2351x; 1.2586x over previous
import functools

import jax
import jax.numpy as jnp
import numpy as np
from jax import lax
from jax.experimental import pallas as pl
from jax.experimental.pallas import tpu as pltpu
from jax.experimental.pallas import tpu_sc as plsc

N = 10000
E = 320000
HID = 128
H = 8
DH = 16
FFN = 256
L = 2

NW = 32
CH = 128
E2 = E + N
PER_W = 10368
E2P = PER_W * NW
NCHUNK = PER_W // CH
NP = 10240
RPT = NP // 16

R = 1000
GRID = N // R
RE = 1024
GRIDE = E2P // RE

NF = 9
NT = 3
NP2 = 10368
NTH = NP2 // NT
TRASH = NTH
ACCR = NTH + 8
NCH2 = E2P // CH
NSLOT = 4


def _dot(a, b):
    return lax.dot_general(a, b, (((1,), (0,)), ((), ())),
                           precision=lax.Precision.HIGHEST,
                           preferred_element_type=jnp.float32)


def _ln(h, g, b):
    mu = jnp.mean(h, axis=1, keepdims=True)
    d = h - mu
    var = jnp.mean(d * d, axis=1, keepdims=True)
    return d * lax.rsqrt(var + 1e-5) * g + b


def _in_body(x_ref, w_ref, b_ref, o_ref):
    o_ref[...] = _dot(x_ref[...], w_ref[...]) + b_ref[...]


def _qkv_body(h_ref, g_ref, b_ref, w_ref, bias_ref, q_ref, kv_ref):
    hn = _ln(h_ref[...], g_ref[...], b_ref[...])
    qkv = _dot(hn, w_ref[...]) + bias_ref[...]
    q_ref[...] = qkv[:, :HID]
    kv_ref[...] = qkv[:, HID:]


def _edge_body(dst_ref, qd_ref, kvg_ref, b1_ref, b2_ref, out_ref,
               wvs_ref, acc_ref):
    j = pl.program_id(0)

    @pl.when(j == 0)
    def _():
        acc_ref[...] = jnp.zeros((NSLOT, NP2, NF * 16), jnp.float32)

    rid = j * RE + lax.broadcasted_iota(jnp.int32, (RE, 1), 0)
    qd = qd_ref[...]
    kvg = kvg_ref[...]
    scores = _dot(qd * kvg[:, :HID], b1_ref[...])
    e = jnp.exp(scores) * jnp.where(rid < E2, 1.0, 0.0)
    erep = _dot(e, b2_ref[...])
    realf = jnp.where(rid < E, 1.0, 0.0)
    wvs_ref[...] = jnp.concatenate(
        [kvg[:, HID:] * erep, e, realf,
         jnp.zeros((RE, 7), jnp.float32)], axis=1)

    def body(kb, carry):
        for s in range(NSLOT):
            k = kb * NSLOT + s
            d = dst_ref[0, 0, k]
            acc_ref[s, pl.ds(d, 1), :] = (acc_ref[s, pl.ds(d, 1), :]
                                          + wvs_ref[pl.ds(k, 1), :])
        return carry

    lax.fori_loop(0, RE // NSLOT, body, 0)

    @pl.when(j == GRIDE - 1)
    def _():
        tot = acc_ref[0]
        for s in range(1, NSLOT):
            tot = tot + acc_ref[s]
        out_ref[...] = tot


def _post_body(h_ref, n_ref, s_ref, wo_ref, bo_ref,
               g2_ref, b2_ref, wf1_ref, bf1_ref, wf2_ref, bf2_ref, o_ref):
    num = n_ref[...]
    s = s_ref[...]
    rec = jnp.concatenate(
        [jnp.broadcast_to(s[:, h:h + 1], (R, DH)) for h in range(H)], axis=1)
    deg = s[:, 8:9]
    agg = jnp.where(deg > 0.0, num / rec, 0.0)
    h2 = h_ref[...] + _dot(agg, wo_ref[...]) + bo_ref[...]
    hn = _ln(h2, g2_ref[...], b2_ref[...])
    f = _dot(hn, wf1_ref[...]) + bf1_ref[...]
    f = 0.5 * f * (1.0 + lax.erf(f * 0.7071067811865476))
    o_ref[...] = h2 + _dot(f, wf2_ref[...]) + bf2_ref[...]


def _row_spec(rows, cols):
    return pl.BlockSpec((rows, cols), lambda i: (i, 0))


def _full_spec(shape):
    nd = len(shape)
    return pl.BlockSpec(shape, lambda i: (0,) * nd)


def _call_in(x, wT, b2):
    return pl.pallas_call(
        _in_body,
        grid=(GRID,),
        in_specs=[_row_spec(R, HID), _full_spec((HID, HID)),
                  _full_spec((1, HID))],
        out_specs=_row_spec(R, HID),
        out_shape=jax.ShapeDtypeStruct((N, HID), jnp.float32),
    )(x, wT, b2)


def _call_qkv(h, g, b, wT, bias):
    return pl.pallas_call(
        _qkv_body,
        grid=(GRID,),
        in_specs=[_row_spec(R, HID), _full_spec((1, HID)), _full_spec((1, HID)),
                  _full_spec((HID, 3 * HID)), _full_spec((1, 3 * HID))],
        out_specs=[_row_spec(R, HID), _row_spec(R, 2 * HID)],
        out_shape=[jax.ShapeDtypeStruct((N, HID), jnp.float32),
                   jax.ShapeDtypeStruct((N, 2 * HID), jnp.float32)],
    )(h, g, b, wT, bias)


def _call_edge_compute(dst3, qd, kvg, b1m, b2m):
    return pl.pallas_call(
        _edge_body,
        grid=(GRIDE,),
        in_specs=[pl.BlockSpec((1, 1, RE), lambda j: (j, 0, 0),
                               memory_space=pltpu.SMEM),
                  _row_spec(RE, HID), _row_spec(RE, 2 * HID),
                  _full_spec((HID, H)), _full_spec((H, HID))],
        out_specs=_full_spec((NP2, NF * 16)),
        out_shape=jax.ShapeDtypeStruct((NP2, NF * 16), jnp.float32),
        scratch_shapes=[pltpu.VMEM((RE, NF * 16), jnp.float32),
                        pltpu.VMEM((NSLOT, NP2, NF * 16), jnp.float32)],
    )(dst3, qd, kvg, b1m, b2m)


def _call_post(h, n, s, woT, bo, g2, b2, wf1T, bf1, wf2T, bf2):
    return pl.pallas_call(
        _post_body,
        grid=(GRID,),
        in_specs=[_row_spec(R, HID), _row_spec(R, HID),
                  _row_spec(R, 16),
                  _full_spec((HID, HID)), _full_spec((1, HID)),
                  _full_spec((1, HID)), _full_spec((1, HID)),
                  _full_spec((HID, FFN)), _full_spec((1, FFN)),
                  _full_spec((FFN, HID)), _full_spec((1, HID))],
        out_specs=_row_spec(R, HID),
        out_shape=jax.ShapeDtypeStruct((N, HID), jnp.float32),
    )(h, n, s, woT, bo, g2, b2, wf1T, bf1, wf2T, bf2)


def _make_gather_kernel():
    mesh = plsc.VectorSubcoreMesh(core_axis_name="c", subcore_axis_name="s")

    @functools.partial(
        pl.kernel, mesh=mesh,
        out_type=[jax.ShapeDtypeStruct((E2P, HID), jnp.float32),
                  jax.ShapeDtypeStruct((E2P, 2 * HID), jnp.float32)],
        scratch_types=[
            pltpu.VMEM((CH,), jnp.int32),
            pltpu.VMEM((CH,), jnp.int32),
            pltpu.VMEM((CH, HID), jnp.float32),
            pltpu.VMEM((CH, 2 * HID), jnp.float32),
            pltpu.SemaphoreType.DMA,
            pltpu.SemaphoreType.DMA,
        ])
    def gather_kernel(q_hbm, kv_hbm, src_hbm, dst_hbm, qd_out, kvg_out,
                      srcb, dstb, qrows, kvrows, sem1, sem2):
        cid = lax.axis_index("c")
        sid = lax.axis_index("s")
        wid = sid * 2 + cid
        ebase = wid * PER_W

        def chunk(t, carry):
            base = ebase + t * CH
            pltpu.sync_copy(src_hbm.at[pl.ds(base, CH)], srcb)
            pltpu.sync_copy(dst_hbm.at[pl.ds(base, CH)], dstb)
            cq = pltpu.async_copy(q_hbm.at[dstb], qrows, sem1)
            ck = pltpu.async_copy(kv_hbm.at[srcb], kvrows, sem2)
            cq.wait()
            ck.wait()
            pltpu.sync_copy(qrows, qd_out.at[pl.ds(base, CH)])
            pltpu.sync_copy(kvrows, kvg_out.at[pl.ds(base, CH)])
            return carry

        lax.fori_loop(0, NCHUNK, chunk, 0)

    return gather_kernel


_gather_kernel = _make_gather_kernel()


def kernel(x, edge_index, Win, b_in, Wq, bq, Wk, bk, Wv, bv, Wo, bo,
           g1, b1, g2, b2, Wf1, bf1, Wf2, bf2):
    src = edge_index[0]
    dst = edge_index[1]
    idxN = jnp.arange(N, dtype=jnp.int32)
    pad = jnp.zeros((E2P - E2,), jnp.int32)
    src2 = jnp.concatenate([src, idxN, pad])
    dst2 = jnp.concatenate([dst, idxN, pad])
    blk = (np.arange(HID)[:, None] // DH) == np.arange(H)[None, :]
    b1m = jnp.asarray(blk, jnp.float32)
    b2m = jnp.asarray(blk.T, jnp.float32)

    h = _call_in(x, Win.T, b_in[None, :])
    inv = 1.0 / jnp.sqrt(float(DH))
    dst3 = dst2.reshape(GRIDE, 1, RE)
    for l in range(L):
        wqkvT = jnp.concatenate(
            [Wq[l].T * inv, Wk[l].T, Wv[l].T], axis=1)
        bqkv = jnp.concatenate([bq[l] * inv, bk[l], bv[l]])[None, :]
        q, kv = _call_qkv(h, g1[l][None, :], b1[l][None, :], wqkvT, bqkv)
        qd, kvg = _gather_kernel(q, kv, src2, dst2)
        acc2 = _call_edge_compute(dst3, qd, kvg, b1m, b2m)
        num = acc2[:N, :HID]
        ss = acc2[:N, HID:HID + 16]
        h = _call_post(h, num, ss,
                       Wo[l].T, bo[l][None, :], g2[l][None, :], b2[l][None, :],
                       Wf1[l].T, bf1[l][None, :], Wf2[l].T, bf2[l][None, :])
    return h


# speedup vs baseline: 16.3123x; 1.0707x over previous
import functools

import jax
import jax.numpy as jnp
import numpy as np
from jax import lax
from jax.experimental import pallas as pl
from jax.experimental.pallas import tpu as pltpu
from jax.experimental.pallas import tpu_sc as plsc

N = 10000
E = 320000
HID = 128
H = 8
DH = 16
FFN = 256
L = 2

NW = 32
CH = 128
E2 = E + N
PER_W = 10368
E2P = PER_W * NW
NCHUNK = PER_W // CH
NP = 10240
RPT = NP // 16

R = 1000
GRID = N // R
RE = 1024
GRIDE = E2P // RE

NF = 9
NT = 3
NP2 = 10368
NTH = NP2 // NT
TRASH = NTH
ACCR = NTH + 8
NCH2 = E2P // CH
NSLOT = 4
UNROLL = 8


def _dot(a, b):
    return lax.dot_general(a, b, (((1,), (0,)), ((), ())),
                           precision=lax.Precision.HIGHEST,
                           preferred_element_type=jnp.float32)


def _ln(h, g, b):
    mu = jnp.mean(h, axis=1, keepdims=True)
    d = h - mu
    var = jnp.mean(d * d, axis=1, keepdims=True)
    return d * lax.rsqrt(var + 1e-5) * g + b


def _in_body(x_ref, w_ref, b_ref, o_ref):
    o_ref[...] = _dot(x_ref[...], w_ref[...]) + b_ref[...]


def _qkv_body(h_ref, g_ref, b_ref, w_ref, bias_ref, q_ref, kv_ref):
    hn = _ln(h_ref[...], g_ref[...], b_ref[...])
    qkv = _dot(hn, w_ref[...]) + bias_ref[...]
    q_ref[...] = qkv[:, :HID]
    kv_ref[...] = qkv[:, HID:]


def _edge_body(dst_ref, qd_ref, kvg_ref, b1_ref, b2_ref, out_ref,
               wvs_ref, acc_ref):
    j = pl.program_id(0)

    @pl.when(j == 0)
    def _():
        acc_ref[...] = jnp.zeros((NSLOT, NP2, NF * 16), jnp.float32)

    rid = j * RE + lax.broadcasted_iota(jnp.int32, (RE, 1), 0)
    qd = qd_ref[...]
    kvg = kvg_ref[...]
    scores = _dot(qd * kvg[:, :HID], b1_ref[...])
    e = jnp.exp(scores) * jnp.where(rid < E2, 1.0, 0.0)
    erep = _dot(e, b2_ref[...])
    realf = jnp.where(rid < E, 1.0, 0.0)
    wvs_ref[...] = jnp.concatenate(
        [kvg[:, HID:] * erep, e, realf,
         jnp.zeros((RE, 7), jnp.float32)], axis=1)

    def body(kb, carry):
        for u in range(UNROLL):
            s = u % NSLOT
            k = kb * UNROLL + u
            d = dst_ref[0, 0, k]
            acc_ref[s, pl.ds(d, 1), :] = (acc_ref[s, pl.ds(d, 1), :]
                                          + wvs_ref[pl.ds(k, 1), :])
        return carry

    lax.fori_loop(0, RE // UNROLL, body, 0)

    @pl.when(j == GRIDE - 1)
    def _():
        tot = acc_ref[0]
        for s in range(1, NSLOT):
            tot = tot + acc_ref[s]
        out_ref[...] = tot


def _post_body(h_ref, n_ref, s_ref, wo_ref, bo_ref,
               g2_ref, b2_ref, wf1_ref, bf1_ref, wf2_ref, bf2_ref, o_ref):
    num = n_ref[...]
    s = s_ref[...]
    rec = jnp.concatenate(
        [jnp.broadcast_to(s[:, h:h + 1], (R, DH)) for h in range(H)], axis=1)
    deg = s[:, 8:9]
    agg = jnp.where(deg > 0.0, num / rec, 0.0)
    h2 = h_ref[...] + _dot(agg, wo_ref[...]) + bo_ref[...]
    hn = _ln(h2, g2_ref[...], b2_ref[...])
    f = _dot(hn, wf1_ref[...]) + bf1_ref[...]
    f = 0.5 * f * (1.0 + lax.erf(f * 0.7071067811865476))
    o_ref[...] = h2 + _dot(f, wf2_ref[...]) + bf2_ref[...]


def _row_spec(rows, cols):
    return pl.BlockSpec((rows, cols), lambda i: (i, 0))


def _full_spec(shape):
    nd = len(shape)
    return pl.BlockSpec(shape, lambda i: (0,) * nd)


def _call_in(x, wT, b2):
    return pl.pallas_call(
        _in_body,
        grid=(GRID,),
        in_specs=[_row_spec(R, HID), _full_spec((HID, HID)),
                  _full_spec((1, HID))],
        out_specs=_row_spec(R, HID),
        out_shape=jax.ShapeDtypeStruct((N, HID), jnp.float32),
    )(x, wT, b2)


def _call_qkv(h, g, b, wT, bias):
    return pl.pallas_call(
        _qkv_body,
        grid=(GRID,),
        in_specs=[_row_spec(R, HID), _full_spec((1, HID)), _full_spec((1, HID)),
                  _full_spec((HID, 3 * HID)), _full_spec((1, 3 * HID))],
        out_specs=[_row_spec(R, HID), _row_spec(R, 2 * HID)],
        out_shape=[jax.ShapeDtypeStruct((N, HID), jnp.float32),
                   jax.ShapeDtypeStruct((N, 2 * HID), jnp.float32)],
    )(h, g, b, wT, bias)


def _call_edge_compute(dst3, qd, kvg, b1m, b2m):
    return pl.pallas_call(
        _edge_body,
        grid=(GRIDE,),
        in_specs=[pl.BlockSpec((1, 1, RE), lambda j: (j, 0, 0),
                               memory_space=pltpu.SMEM),
                  _row_spec(RE, HID), _row_spec(RE, 2 * HID),
                  _full_spec((HID, H)), _full_spec((H, HID))],
        out_specs=_full_spec((NP2, NF * 16)),
        out_shape=jax.ShapeDtypeStruct((NP2, NF * 16), jnp.float32),
        scratch_shapes=[pltpu.VMEM((RE, NF * 16), jnp.float32),
                        pltpu.VMEM((NSLOT, NP2, NF * 16), jnp.float32)],
    )(dst3, qd, kvg, b1m, b2m)


def _call_post(h, n, s, woT, bo, g2, b2, wf1T, bf1, wf2T, bf2):
    return pl.pallas_call(
        _post_body,
        grid=(GRID,),
        in_specs=[_row_spec(R, HID), _row_spec(R, HID),
                  _row_spec(R, 16),
                  _full_spec((HID, HID)), _full_spec((1, HID)),
                  _full_spec((1, HID)), _full_spec((1, HID)),
                  _full_spec((HID, FFN)), _full_spec((1, FFN)),
                  _full_spec((FFN, HID)), _full_spec((1, HID))],
        out_specs=_row_spec(R, HID),
        out_shape=jax.ShapeDtypeStruct((N, HID), jnp.float32),
    )(h, n, s, woT, bo, g2, b2, wf1T, bf1, wf2T, bf2)


def _make_gather_kernel():
    mesh = plsc.VectorSubcoreMesh(core_axis_name="c", subcore_axis_name="s")

    @functools.partial(
        pl.kernel, mesh=mesh,
        out_type=[jax.ShapeDtypeStruct((E2P, HID), jnp.float32),
                  jax.ShapeDtypeStruct((E2P, 2 * HID), jnp.float32)],
        scratch_types=[
            pltpu.VMEM((CH,), jnp.int32),
            pltpu.VMEM((CH,), jnp.int32),
            pltpu.VMEM((CH, HID), jnp.float32),
            pltpu.VMEM((CH, 2 * HID), jnp.float32),
            pltpu.SemaphoreType.DMA,
            pltpu.SemaphoreType.DMA,
        ])
    def gather_kernel(q_hbm, kv_hbm, src_hbm, dst_hbm, qd_out, kvg_out,
                      srcb, dstb, qrows, kvrows, sem1, sem2):
        cid = lax.axis_index("c")
        sid = lax.axis_index("s")
        wid = sid * 2 + cid
        ebase = wid * PER_W

        def chunk(t, carry):
            base = ebase + t * CH
            pltpu.sync_copy(src_hbm.at[pl.ds(base, CH)], srcb)
            pltpu.sync_copy(dst_hbm.at[pl.ds(base, CH)], dstb)
            cq = pltpu.async_copy(q_hbm.at[dstb], qrows, sem1)
            ck = pltpu.async_copy(kv_hbm.at[srcb], kvrows, sem2)
            cq.wait()
            ck.wait()
            pltpu.sync_copy(qrows, qd_out.at[pl.ds(base, CH)])
            pltpu.sync_copy(kvrows, kvg_out.at[pl.ds(base, CH)])
            return carry

        lax.fori_loop(0, NCHUNK, chunk, 0)

    return gather_kernel


_gather_kernel = _make_gather_kernel()


def kernel(x, edge_index, Win, b_in, Wq, bq, Wk, bk, Wv, bv, Wo, bo,
           g1, b1, g2, b2, Wf1, bf1, Wf2, bf2):
    src = edge_index[0]
    dst = edge_index[1]
    idxN = jnp.arange(N, dtype=jnp.int32)
    pad = jnp.zeros((E2P - E2,), jnp.int32)
    src2 = jnp.concatenate([src, idxN, pad])
    dst2 = jnp.concatenate([dst, idxN, pad])
    blk = (np.arange(HID)[:, None] // DH) == np.arange(H)[None, :]
    b1m = jnp.asarray(blk, jnp.float32)
    b2m = jnp.asarray(blk.T, jnp.float32)

    h = _call_in(x, Win.T, b_in[None, :])
    inv = 1.0 / jnp.sqrt(float(DH))
    dst3 = dst2.reshape(GRIDE, 1, RE)
    for l in range(L):
        wqkvT = jnp.concatenate(
            [Wq[l].T * inv, Wk[l].T, Wv[l].T], axis=1)
        bqkv = jnp.concatenate([bq[l] * inv, bk[l], bv[l]])[None, :]
        q, kv = _call_qkv(h, g1[l][None, :], b1[l][None, :], wqkvT, bqkv)
        qd, kvg = _gather_kernel(q, kv, src2, dst2)
        acc2 = _call_edge_compute(dst3, qd, kvg, b1m, b2m)
        num = acc2[:N, :HID]
        ss = acc2[:N, HID:HID + 16]
        h = _call_post(h, num, ss,
                       Wo[l].T, bo[l][None, :], g2[l][None, :], b2[l][None, :],
                       Wf1[l].T, bf1[l][None, :], Wf2[l].T, bf2[l][None, :])
    return h


# speedup vs baseline: 17.1450x; 1.0510x over previous
import functools

import jax
import jax.numpy as jnp
import numpy as np
from jax import lax
from jax.experimental import pallas as pl
from jax.experimental.pallas import tpu as pltpu
from jax.experimental.pallas import tpu_sc as plsc

N = 10000
E = 320000
HID = 128
H = 8
DH = 16
FFN = 256
L = 2

NW = 32
CH = 128
E2 = E + N
PER_W = 10368
E2P = PER_W * NW
NCHUNK = PER_W // CH
NP = 10240
RPT = NP // 16

R = 1000
GRID = N // R
RE = 1024
GRIDE = E2P // RE

NF = 9
NT = 3
NP2 = 10368
NTH = NP2 // NT
TRASH = NTH
ACCR = NTH + 8
NCH2 = E2P // CH
NSLOT = 4
UNROLL = 8


def _dot(a, b):
    return lax.dot_general(a, b, (((1,), (0,)), ((), ())),
                           precision=lax.Precision.HIGHEST,
                           preferred_element_type=jnp.float32)


def _ln(h, g, b):
    mu = jnp.mean(h, axis=1, keepdims=True)
    d = h - mu
    var = jnp.mean(d * d, axis=1, keepdims=True)
    return d * lax.rsqrt(var + 1e-5) * g + b


def _in_body(x_ref, w_ref, b_ref, o_ref):
    o_ref[...] = _dot(x_ref[...], w_ref[...]) + b_ref[...]


def _qkv_body(h_ref, g_ref, b_ref, w_ref, bias_ref, q_ref, kv_ref):
    hn = _ln(h_ref[...], g_ref[...], b_ref[...])
    qkv = _dot(hn, w_ref[...]) + bias_ref[...]
    q_ref[...] = qkv[:, :HID]
    kv_ref[...] = qkv[:, HID:]


def _edge_body(dst_ref, qd_ref, kvg_ref, b1_ref, b2_ref, out_ref,
               wvs_ref, acc_ref):
    j = pl.program_id(0)

    @pl.when(j == 0)
    def _():
        acc_ref[...] = jnp.zeros((NSLOT, NP2, NF * 16), jnp.float32)

    rid = j * RE + lax.broadcasted_iota(jnp.int32, (RE, 1), 0)
    qd = qd_ref[...]
    kvg = kvg_ref[...]
    scores = _dot(qd * kvg[:, :HID], b1_ref[...])
    e = jnp.exp(scores) * jnp.where(rid < E2, 1.0, 0.0)
    erep = _dot(e, b2_ref[...])
    realf = jnp.where(rid < E, 1.0, 0.0)
    wvs_ref[...] = jnp.concatenate(
        [kvg[:, HID:] * erep, e, realf,
         jnp.zeros((RE, 7), jnp.float32)], axis=1)

    def body(kb, carry):
        for u in range(UNROLL):
            s = u % NSLOT
            k = kb * UNROLL + u
            d = dst_ref[0, 0, k]
            acc_ref[s, pl.ds(d, 1), :] = (acc_ref[s, pl.ds(d, 1), :]
                                          + wvs_ref[pl.ds(k, 1), :])
        return carry

    lax.fori_loop(0, RE // UNROLL, body, 0)

    @pl.when(j == GRIDE - 1)
    def _():
        tot = acc_ref[0]
        for s in range(1, NSLOT):
            tot = tot + acc_ref[s]
        out_ref[...] = tot


def _post_body(h_ref, n_ref, s_ref, wo_ref, bo_ref,
               g2_ref, b2_ref, wf1_ref, bf1_ref, wf2_ref, bf2_ref, o_ref):
    num = n_ref[...]
    s = s_ref[...]
    rec = jnp.concatenate(
        [jnp.broadcast_to(s[:, h:h + 1], (R, DH)) for h in range(H)], axis=1)
    deg = s[:, 8:9]
    agg = jnp.where(deg > 0.0, num / rec, 0.0)
    h2 = h_ref[...] + _dot(agg, wo_ref[...]) + bo_ref[...]
    hn = _ln(h2, g2_ref[...], b2_ref[...])
    f = _dot(hn, wf1_ref[...]) + bf1_ref[...]
    f = 0.5 * f * (1.0 + lax.erf(f * 0.7071067811865476))
    o_ref[...] = h2 + _dot(f, wf2_ref[...]) + bf2_ref[...]


def _row_spec(rows, cols):
    return pl.BlockSpec((rows, cols), lambda i: (i, 0))


def _full_spec(shape):
    nd = len(shape)
    return pl.BlockSpec(shape, lambda i: (0,) * nd)


def _call_in(x, wT, b2):
    return pl.pallas_call(
        _in_body,
        grid=(GRID,),
        in_specs=[_row_spec(R, HID), _full_spec((HID, HID)),
                  _full_spec((1, HID))],
        out_specs=_row_spec(R, HID),
        out_shape=jax.ShapeDtypeStruct((N, HID), jnp.float32),
    )(x, wT, b2)


def _call_qkv(h, g, b, wT, bias):
    return pl.pallas_call(
        _qkv_body,
        grid=(GRID,),
        in_specs=[_row_spec(R, HID), _full_spec((1, HID)), _full_spec((1, HID)),
                  _full_spec((HID, 3 * HID)), _full_spec((1, 3 * HID))],
        out_specs=[_row_spec(R, HID), _row_spec(R, 2 * HID)],
        out_shape=[jax.ShapeDtypeStruct((N, HID), jnp.float32),
                   jax.ShapeDtypeStruct((N, 2 * HID), jnp.float32)],
    )(h, g, b, wT, bias)


def _call_edge_compute(dst3, qd, kvg, b1m, b2m):
    return pl.pallas_call(
        _edge_body,
        grid=(GRIDE,),
        in_specs=[pl.BlockSpec((1, 1, RE), lambda j: (j, 0, 0),
                               memory_space=pltpu.SMEM),
                  _row_spec(RE, HID), _row_spec(RE, 2 * HID),
                  _full_spec((HID, H)), _full_spec((H, HID))],
        out_specs=_full_spec((NP2, NF * 16)),
        out_shape=jax.ShapeDtypeStruct((NP2, NF * 16), jnp.float32),
        scratch_shapes=[pltpu.VMEM((RE, NF * 16), jnp.float32),
                        pltpu.VMEM((NSLOT, NP2, NF * 16), jnp.float32)],
    )(dst3, qd, kvg, b1m, b2m)


def _call_post(h, n, s, woT, bo, g2, b2, wf1T, bf1, wf2T, bf2):
    return pl.pallas_call(
        _post_body,
        grid=(GRID,),
        in_specs=[_row_spec(R, HID), _row_spec(R, HID),
                  _row_spec(R, 16),
                  _full_spec((HID, HID)), _full_spec((1, HID)),
                  _full_spec((1, HID)), _full_spec((1, HID)),
                  _full_spec((HID, FFN)), _full_spec((1, FFN)),
                  _full_spec((FFN, HID)), _full_spec((1, HID))],
        out_specs=_row_spec(R, HID),
        out_shape=jax.ShapeDtypeStruct((N, HID), jnp.float32),
    )(h, n, s, woT, bo, g2, b2, wf1T, bf1, wf2T, bf2)


def _make_gather_kernel():
    mesh = plsc.VectorSubcoreMesh(core_axis_name="c", subcore_axis_name="s")

    @functools.partial(
        pl.kernel, mesh=mesh,
        out_type=[jax.ShapeDtypeStruct((E2P, HID), jnp.float32),
                  jax.ShapeDtypeStruct((E2P, 2 * HID), jnp.float32)],
        scratch_types=[
            pltpu.VMEM((CH,), jnp.int32),
            pltpu.VMEM((CH,), jnp.int32),
            pltpu.VMEM((CH,), jnp.int32),
            pltpu.VMEM((CH,), jnp.int32),
            pltpu.VMEM((CH, HID), jnp.float32),
            pltpu.VMEM((CH, HID), jnp.float32),
            pltpu.VMEM((CH, 2 * HID), jnp.float32),
            pltpu.VMEM((CH, 2 * HID), jnp.float32),
            pltpu.SemaphoreType.DMA,
            pltpu.SemaphoreType.DMA,
            pltpu.SemaphoreType.DMA,
            pltpu.SemaphoreType.DMA,
        ])
    def gather_kernel(q_hbm, kv_hbm, src_hbm, dst_hbm, qd_out, kvg_out,
                      srcb0, srcb1, dstb0, dstb1, qrows0, qrows1,
                      kvrows0, kvrows1, semq0, semq1, semk0, semk1):
        cid = lax.axis_index("c")
        sid = lax.axis_index("s")
        wid = sid * 2 + cid
        ebase = wid * PER_W
        srcb = (srcb0, srcb1)
        dstb = (dstb0, dstb1)
        qrows = (qrows0, qrows1)
        kvrows = (kvrows0, kvrows1)
        semq = (semq0, semq1)
        semk = (semk0, semk1)

        def issue(tt, pp):
            base = ebase + tt * CH
            pltpu.sync_copy(src_hbm.at[pl.ds(base, CH)], srcb[pp])
            pltpu.sync_copy(dst_hbm.at[pl.ds(base, CH)], dstb[pp])
            cq = pltpu.async_copy(q_hbm.at[dstb[pp]], qrows[pp], semq[pp])
            ck = pltpu.async_copy(kv_hbm.at[srcb[pp]], kvrows[pp], semk[pp])
            return cq, ck

        def wait_and_flush(tt, pp):
            pltpu.make_async_copy(q_hbm.at[dstb[pp]], qrows[pp],
                                  semq[pp]).wait()
            pltpu.make_async_copy(kv_hbm.at[srcb[pp]], kvrows[pp],
                                  semk[pp]).wait()
            base = ebase + tt * CH
            pltpu.sync_copy(qrows[pp], qd_out.at[pl.ds(base, CH)])
            pltpu.sync_copy(kvrows[pp], kvg_out.at[pl.ds(base, CH)])

        issue(0, 0)

        def pair(t2, carry):
            a = 1 + 2 * t2
            issue(a, 1)
            wait_and_flush(a - 1, 0)
            issue(a + 1, 0)
            wait_and_flush(a, 1)
            return carry

        lax.fori_loop(0, (NCHUNK - 1) // 2, pair, 0)
        wait_and_flush(NCHUNK - 1, 0)

    return gather_kernel


_gather_kernel = _make_gather_kernel()


def kernel(x, edge_index, Win, b_in, Wq, bq, Wk, bk, Wv, bv, Wo, bo,
           g1, b1, g2, b2, Wf1, bf1, Wf2, bf2):
    src = edge_index[0]
    dst = edge_index[1]
    idxN = jnp.arange(N, dtype=jnp.int32)
    pad = jnp.zeros((E2P - E2,), jnp.int32)
    src2 = jnp.concatenate([src, idxN, pad])
    dst2 = jnp.concatenate([dst, idxN, pad])
    blk = (np.arange(HID)[:, None] // DH) == np.arange(H)[None, :]
    b1m = jnp.asarray(blk, jnp.float32)
    b2m = jnp.asarray(blk.T, jnp.float32)

    h = _call_in(x, Win.T, b_in[None, :])
    inv = 1.0 / jnp.sqrt(float(DH))
    dst3 = dst2.reshape(GRIDE, 1, RE)
    for l in range(L):
        wqkvT = jnp.concatenate(
            [Wq[l].T * inv, Wk[l].T, Wv[l].T], axis=1)
        bqkv = jnp.concatenate([bq[l] * inv, bk[l], bv[l]])[None, :]
        q, kv = _call_qkv(h, g1[l][None, :], b1[l][None, :], wqkvT, bqkv)
        qd, kvg = _gather_kernel(q, kv, src2, dst2)
        acc2 = _call_edge_compute(dst3, qd, kvg, b1m, b2m)
        num = acc2[:N, :HID]
        ss = acc2[:N, HID:HID + 16]
        h = _call_post(h, num, ss,
                       Wo[l].T, bo[l][None, :], g2[l][None, :], b2[l][None, :],
                       Wf1[l].T, bf1[l][None, :], Wf2[l].T, bf2[l][None, :])
    return h
